# baseline (speedup 1.0000x reference)
import functools
import jax
import jax.numpy as jnp
from jax import lax
from jax.experimental import pallas as pl
from jax.experimental.pallas import tpu as pltpu
from jax.experimental.pallas import tpu_sc as plsc

NUM_WORKERS = 32
LANES = 16


@functools.lru_cache(maxsize=None)
def _make_sc_kernel(T: int, E: int):
    CH = E // NUM_WORKERS
    NSL = CH // LANES

    mesh = plsc.VectorSubcoreMesh(core_axis_name="c", subcore_axis_name="s")

    @functools.partial(
        pl.kernel,
        out_type=jax.ShapeDtypeStruct((T, E), jnp.float32),
        mesh=mesh,
        scratch_types=[
            pltpu.VMEM((CH,), jnp.float32),
            pltpu.VMEM((CH,), jnp.float32),
            pltpu.VMEM((CH,), jnp.float32),
            pltpu.VMEM((CH,), jnp.float32),
            pltpu.VMEM((CH,), jnp.float32),
        ],
    )
    def spike_sc(x_hbm, w_hbm, out_hbm, xbuf, wv, vv, rv, ybuf):
        cid = lax.axis_index("c")
        sid = lax.axis_index("s")
        wid = sid * 2 + cid
        base = wid * CH

        pltpu.sync_copy(w_hbm.at[pl.ds(base, CH)], wv)

        def zinit(i, carry):
            s = pl.ds(i * LANES, LANES)
            z = jnp.zeros((LANES,), jnp.float32)
            vv[s] = z
            rv[s] = z
            return carry

        lax.fori_loop(0, NSL, zinit, 0)

        def tstep(t, carry):
            pltpu.sync_copy(x_hbm.at[t, pl.ds(base, CH)], xbuf)

            def slice_body(i, c2):
                s = pl.ds(i * LANES, LANES)
                x = xbuf[s]
                v = vv[s]
                r = rv[s]
                w = wv[s]
                h = w * r
                v = v + (x + h)
                spike = jnp.where(v >= 1.0, 1.0, 0.0)
                v = v * (1.0 - spike)
                r = spike * (r + 1.0)
                vv[s] = v
                rv[s] = r
                ybuf[s] = spike
                return c2

            lax.fori_loop(0, NSL, slice_body, 0)
            pltpu.sync_copy(ybuf, out_hbm.at[t, pl.ds(base, CH)])
            return carry

        lax.fori_loop(0, T, tstep, 0)

    return spike_sc


def kernel(input, lateral_weight):
    T, N, C, J = input.shape
    E = N * C * J
    x2 = input.reshape(T, E)
    w_full = jnp.broadcast_to(lateral_weight, (N, C, J)).reshape(E)
    out = _make_sc_kernel(T, E)(x2, w_full)
    return out.reshape(T, N, C, J)


# speedup vs baseline: 1.7677x; 1.7677x over previous
import functools
import jax
import jax.numpy as jnp
from jax import lax
from jax.experimental import pallas as pl
from jax.experimental.pallas import tpu as pltpu
from jax.experimental.pallas import tpu_sc as plsc

NUM_WORKERS = 32
LANES = 16
K = 8


@functools.lru_cache(maxsize=None)
def _make_sc_kernel(T: int, E: int):
    CH = E // NUM_WORKERS
    NSL = CH // LANES
    NG = T // K

    mesh = plsc.VectorSubcoreMesh(core_axis_name="c", subcore_axis_name="s")

    @functools.partial(
        pl.kernel,
        out_type=jax.ShapeDtypeStruct((T, E), jnp.float32),
        mesh=mesh,
        scratch_types=[
            pltpu.VMEM((K, CH), jnp.float32),
            pltpu.VMEM((K, CH), jnp.float32),
            pltpu.VMEM((K, CH), jnp.float32),
            pltpu.VMEM((K, CH), jnp.float32),
            pltpu.VMEM((CH,), jnp.float32),
            pltpu.VMEM((CH,), jnp.float32),
            pltpu.VMEM((CH,), jnp.float32),
            pltpu.SemaphoreType.DMA,
            pltpu.SemaphoreType.DMA,
            pltpu.SemaphoreType.DMA,
            pltpu.SemaphoreType.DMA,
        ],
    )
    def spike_sc(x_hbm, w_hbm, out_hbm, xb0, xb1, yb0, yb1, wv, vv, hv,
                 si0, si1, so0, so1):
        cid = lax.axis_index("c")
        sid = lax.axis_index("s")
        wid = sid * 2 + cid
        base = wid * CH

        xbufs = [xb0, xb1]
        ybufs = [yb0, yb1]
        sins = [si0, si1]
        souts = [so0, so1]

        din = [None] * NG
        dout = [None] * NG
        din[0] = pltpu.async_copy(
            x_hbm.at[pl.ds(0, K), pl.ds(base, CH)], xb0, si0)
        din[1] = pltpu.async_copy(
            x_hbm.at[pl.ds(K, K), pl.ds(base, CH)], xb1, si1)

        pltpu.sync_copy(w_hbm.at[pl.ds(base, CH)], wv)

        @plsc.parallel_loop(0, NSL, unroll=2)
        def _init(i):
            s = pl.ds(i * LANES, LANES)
            z = jnp.zeros((LANES,), jnp.float32)
            vv[s] = z
            hv[s] = z
            yb1[K - 1, s] = z

        for g in range(NG):
            b = g & 1
            xb = xbufs[b]
            yb = ybufs[b]
            ypb = ybufs[1 - b]
            din[g].wait()
            if g >= 2:
                dout[g - 2].wait()

            @plsc.parallel_loop(0, NSL, unroll=2)
            def _block(i, xb=xb, yb=yb, ypb=ypb):
                s = pl.ds(i * LANES, LANES)
                v = vv[s]
                h = hv[s]
                w = wv[s]
                m = ypb[K - 1, s] >= 0.5
                for k in range(K):
                    h = jnp.where(m, h + w, 0.0)
                    v = v + (xb[k, s] + h)
                    m = v >= 1.0
                    yb[k, s] = jnp.where(m, 1.0, 0.0)
                    v = jnp.where(m, 0.0, v)
                vv[s] = v
                hv[s] = h

            dout[g] = pltpu.async_copy(
                yb, out_hbm.at[pl.ds(g * K, K), pl.ds(base, CH)], souts[b])
            if g + 2 < NG:
                din[g + 2] = pltpu.async_copy(
                    x_hbm.at[pl.ds((g + 2) * K, K), pl.ds(base, CH)],
                    xb, sins[b])

        dout[NG - 2].wait()
        dout[NG - 1].wait()

    return spike_sc


def kernel(input, lateral_weight):
    T, N, C, J = input.shape
    E = N * C * J
    x2 = input.reshape(T, E)
    w_full = jnp.broadcast_to(lateral_weight, (N, C, J)).reshape(E)
    out = _make_sc_kernel(T, E)(x2, w_full)
    return out.reshape(T, N, C, J)


# speedup vs baseline: 6.1680x; 3.4893x over previous
import functools
import jax
import jax.numpy as jnp
from jax import lax
from jax.experimental import pallas as pl
from jax.experimental.pallas import tpu as pltpu
from jax.experimental.pallas import tpu_sc as plsc

NUM_WORKERS = 32
LANES = 16
K = 8


@functools.lru_cache(maxsize=None)
def _make_sc_kernel(T: int, N: int, C: int, J: int):
    assert N == NUM_WORKERS
    CH = C * J
    NSL = CH // LANES
    NG = T // K

    mesh = plsc.VectorSubcoreMesh(core_axis_name="c", subcore_axis_name="s")

    @functools.partial(
        pl.kernel,
        out_type=jax.ShapeDtypeStruct((T, N, C, J), jnp.float32),
        mesh=mesh,
        scratch_types=[
            pltpu.VMEM((K, C, J), jnp.float32),
            pltpu.VMEM((K, C, J), jnp.float32),
            pltpu.VMEM((K, C, J), jnp.float32),
            pltpu.VMEM((K, C, J), jnp.float32),
            pltpu.VMEM((J,), jnp.float32),
            pltpu.VMEM((CH,), jnp.float32),
            pltpu.VMEM((CH,), jnp.float32),
            pltpu.SemaphoreType.DMA,
            pltpu.SemaphoreType.DMA,
            pltpu.SemaphoreType.DMA,
            pltpu.SemaphoreType.DMA,
        ],
    )
    def spike_sc(x_hbm, w_hbm, out_hbm, xb0, xb1, yb0, yb1, wv, vv, hv,
                 si0, si1, so0, so1):
        cid = lax.axis_index("c")
        sid = lax.axis_index("s")
        n = sid * 2 + cid

        xbufs = [xb0, xb1]
        ybufs = [yb0, yb1]
        sins = [si0, si1]
        souts = [so0, so1]

        din = [None] * NG
        dout = [None] * NG
        din[0] = pltpu.async_copy(x_hbm.at[pl.ds(0, K), n], xb0, si0)
        din[1] = pltpu.async_copy(x_hbm.at[pl.ds(K, K), n], xb1, si1)

        pltpu.sync_copy(w_hbm, wv)

        @plsc.parallel_loop(0, NSL, unroll=2)
        def _init(i):
            s = pl.ds(i * LANES, LANES)
            z = jnp.zeros((LANES,), jnp.float32)
            vv[s] = z
            hv[s] = z
            yb1[K - 1, (i * LANES) // J, pl.ds((i * LANES) % J, LANES)] = z

        for g in range(NG):
            b = g & 1
            xb = xbufs[b]
            yb = ybufs[b]
            ypb = ybufs[1 - b]
            din[g].wait()
            if g >= 2:
                dout[g - 2].wait()

            @plsc.parallel_loop(0, NSL, unroll=2)
            def _block(i, xb=xb, yb=yb, ypb=ypb):
                c = (i * LANES) // J
                s = pl.ds((i * LANES) % J, LANES)
                sf = pl.ds(i * LANES, LANES)
                v = vv[sf]
                h = hv[sf]
                w = wv[s]
                m = ypb[K - 1, c, s] >= 0.5
                for k in range(K):
                    h = jnp.where(m, h + w, 0.0)
                    v = v + (xb[k, c, s] + h)
                    m = v >= 1.0
                    yb[k, c, s] = jnp.where(m, 1.0, 0.0)
                    v = jnp.where(m, 0.0, v)
                vv[sf] = v
                hv[sf] = h

            dout[g] = pltpu.async_copy(
                yb, out_hbm.at[pl.ds(g * K, K), n], souts[b])
            if g + 2 < NG:
                din[g + 2] = pltpu.async_copy(
                    x_hbm.at[pl.ds((g + 2) * K, K), n], xb, sins[b])

        dout[NG - 2].wait()
        dout[NG - 1].wait()

    return spike_sc


def kernel(input, lateral_weight):
    T, N, C, J = input.shape
    return _make_sc_kernel(T, N, C, J)(input, lateral_weight)
